# baseline (speedup 1.0000x reference)
import functools

import jax
import jax.numpy as jnp
from jax import lax
from jax.experimental import pallas as pl
from jax.experimental.pallas import tpu as pltpu
from jax.experimental.pallas import tpu_sc as plsc

B = 16384
D = 64
CHUNK = 128
_info = plsc.get_sparse_core_info()
NC, NS = _info.num_cores, _info.num_subcores
NW = NC * NS
BPW = B // NW
NCH = BPW // CHUNK

_mesh = plsc.VectorSubcoreMesh(core_axis_name="c", subcore_axis_name="s")


@functools.partial(
    pl.kernel,
    mesh=_mesh,
    compiler_params=pltpu.CompilerParams(use_tc_tiling_on_sc=False),
    out_type=jax.ShapeDtypeStruct((4, B, D), jnp.float32),
    scratch_types=[
        pltpu.VMEM((NCH, CHUNK), jnp.int32),
        pltpu.VMEM((BPW, D), jnp.float32),
        pltpu.VMEM((BPW, D), jnp.float32),
        pltpu.SemaphoreType.DMA,
        pltpu.SemaphoreType.DMA,
    ],
)
def _sc_gather4(ct_hbm, dose_hbm, time_hbm, bid_hbm,
                ec_hbm, ed_hbm, et_hbm, eb_hbm,
                out_hbm, idx_v, rows_a, rows_b, gsem, osem):
    wid = lax.axis_index("s") * NC + lax.axis_index("c")
    base = wid * BPW
    cbase = wid * NCH

    idxs = (ct_hbm, dose_hbm, time_hbm, bid_hbm)
    tabs = (ec_hbm, ed_hbm, et_hbm, eb_hbm)
    bufs = (rows_a, rows_b)
    stores = [None, None]
    for j in range(4):
        buf = bufs[j % 2]
        if stores[j % 2] is not None:
            stores[j % 2].wait()
        pltpu.sync_copy(idxs[j].at[pl.ds(cbase, NCH)], idx_v)
        gathers = []
        for k in range(NCH):
            gathers.append(pltpu.async_copy(
                tabs[j].at[idx_v.at[k]], buf.at[pl.ds(k * CHUNK, CHUNK)], gsem))
        for g in gathers:
            g.wait()
        stores[j % 2] = pltpu.async_copy(
            buf, out_hbm.at[j, pl.ds(base, BPW)], osem)
    for s in stores:
        s.wait()


_BS = 2048


def _tc_proj_body(g_ref, w_ref, b_ref, o_ref):
    acc = jnp.dot(g_ref[0], w_ref[0], preferred_element_type=jnp.float32)
    for j in range(1, 4):
        acc = acc + jnp.dot(g_ref[j], w_ref[j], preferred_element_type=jnp.float32)
    o_ref[...] = acc + b_ref[...]


def _tc_proj(g, w4, b2):
    return pl.pallas_call(
        _tc_proj_body,
        grid=(B // _BS,),
        in_specs=[
            pl.BlockSpec((4, _BS, D), lambda i: (0, i, 0)),
            pl.BlockSpec((4, D, D), lambda i: (0, 0, 0)),
            pl.BlockSpec((1, D), lambda i: (0, 0)),
        ],
        out_specs=pl.BlockSpec((_BS, D), lambda i: (i, 0)),
        out_shape=jax.ShapeDtypeStruct((B, D), jnp.float32),
    )(g, w4, b2)


def kernel(cell_type, dose, time, batch_id, E_cell, E_dose, E_time, E_batch, W, b):
    ct = cell_type.astype(jnp.int32).reshape(B // CHUNK, CHUNK)
    do = dose.astype(jnp.int32).reshape(B // CHUNK, CHUNK)
    ti = time.astype(jnp.int32).reshape(B // CHUNK, CHUNK)
    bi = batch_id.astype(jnp.int32).reshape(B // CHUNK, CHUNK)
    g = _sc_gather4(ct, do, ti, bi, E_cell, E_dose, E_time, E_batch)
    w4 = W.reshape(4, D, D)
    return _tc_proj(g, w4, b.reshape(1, D))


# speedup vs baseline: 1.2421x; 1.2421x over previous
import functools

import jax
import jax.numpy as jnp
from jax import lax
from jax.experimental import pallas as pl
from jax.experimental.pallas import tpu as pltpu
from jax.experimental.pallas import tpu_sc as plsc

B = 16384
D = 64
COV_CELL = 1000000
H = 524288
LB = 1024
CHUNK = 128
_info = plsc.get_sparse_core_info()
NC, NS = _info.num_cores, _info.num_subcores
NW = NC * NS
BPW = B // NW
NCH = BPW // CHUNK

_mesh = plsc.VectorSubcoreMesh(core_axis_name="c", subcore_axis_name="s")


def _tc_pairs_body(a_ref, b_ref, o_ref):
    o_ref[...] = jnp.concatenate([a_ref[...].T, b_ref[...].T], axis=1)


def _tc_make_pairs(et2):
    nblk = H // LB
    last = (COV_CELL - 1) // LB
    return pl.pallas_call(
        _tc_pairs_body,
        grid=(nblk,),
        in_specs=[
            pl.BlockSpec((D, LB), lambda g: (0, g)),
            pl.BlockSpec((D, LB), lambda g: (0, jnp.minimum(g + nblk, last))),
        ],
        out_specs=pl.BlockSpec((LB, 2 * D), lambda g: (g, 0)),
        out_shape=jax.ShapeDtypeStruct((H, 2 * D), jnp.float32),
    )(et2, et2)


@functools.partial(
    pl.kernel,
    mesh=_mesh,
    compiler_params=pltpu.CompilerParams(use_tc_tiling_on_sc=True),
    out_type=jax.ShapeDtypeStruct((B, 2 * D), jnp.float32),
    scratch_types=[
        pltpu.VMEM((NCH, CHUNK), jnp.int32),
        pltpu.VMEM((BPW, 2 * D), jnp.float32),
        pltpu.SemaphoreType.DMA,
    ],
)
def _sc_gather_pairs(tab_hbm, idx_hbm, out_hbm, idx_v, rows_v, gsem):
    wid = lax.axis_index("s") * NC + lax.axis_index("c")
    base = wid * BPW
    pltpu.sync_copy(idx_hbm.at[pl.ds(wid * NCH, NCH)], idx_v)
    gathers = []
    for k in range(NCH):
        gathers.append(pltpu.async_copy(
            tab_hbm.at[idx_v.at[k]], rows_v.at[pl.ds(k * CHUNK, CHUNK)], gsem))
    for g in gathers:
        g.wait()
    pltpu.sync_copy(rows_v, out_hbm.at[pl.ds(base, BPW)])


@functools.partial(
    pl.kernel,
    mesh=_mesh,
    compiler_params=pltpu.CompilerParams(use_tc_tiling_on_sc=False),
    out_type=jax.ShapeDtypeStruct((3, B, D), jnp.float32),
    scratch_types=[
        pltpu.VMEM((NCH, CHUNK), jnp.int32),
        pltpu.VMEM((BPW, D), jnp.float32),
        pltpu.VMEM((BPW, D), jnp.float32),
        pltpu.SemaphoreType.DMA,
        pltpu.SemaphoreType.DMA,
    ],
)
def _sc_gather_small(dose_hbm, time_hbm, bid_hbm,
                     ed_hbm, et_hbm, eb_hbm,
                     out_hbm, idx_v, rows_a, rows_b, gsem, osem):
    wid = lax.axis_index("s") * NC + lax.axis_index("c")
    base = wid * BPW
    cbase = wid * NCH

    idxs = (dose_hbm, time_hbm, bid_hbm)
    tabs = (ed_hbm, et_hbm, eb_hbm)
    bufs = (rows_a, rows_b)
    stores = [None, None]
    for j in range(3):
        buf = bufs[j % 2]
        if stores[j % 2] is not None:
            stores[j % 2].wait()
        pltpu.sync_copy(idxs[j].at[pl.ds(cbase, NCH)], idx_v)
        gathers = []
        for k in range(NCH):
            gathers.append(pltpu.async_copy(
                tabs[j].at[idx_v.at[k]], buf.at[pl.ds(k * CHUNK, CHUNK)], gsem))
        for g in gathers:
            g.wait()
        stores[j % 2] = pltpu.async_copy(
            buf, out_hbm.at[j, pl.ds(base, BPW)], osem)
    for s in stores:
        if s is not None:
            s.wait()


_BS = 2048


def _tc_proj_body(gp_ref, pf_ref, gsm_ref, w_ref, b_ref, o_ref):
    g0 = gp_ref[:, 0:D]
    g1 = gp_ref[:, D:2 * D]
    gsel = g0 + pf_ref[...] * (g1 - g0)
    acc = jnp.dot(gsel, w_ref[0], preferred_element_type=jnp.float32)
    for j in range(3):
        acc = acc + jnp.dot(gsm_ref[j], w_ref[j + 1],
                            preferred_element_type=jnp.float32)
    o_ref[...] = acc + b_ref[...]


def _tc_proj(gp, pf, gsm, w4, b2):
    return pl.pallas_call(
        _tc_proj_body,
        grid=(B // _BS,),
        in_specs=[
            pl.BlockSpec((_BS, 2 * D), lambda i: (i, 0)),
            pl.BlockSpec((_BS, 1), lambda i: (i, 0)),
            pl.BlockSpec((3, _BS, D), lambda i: (0, i, 0)),
            pl.BlockSpec((4, D, D), lambda i: (0, 0, 0)),
            pl.BlockSpec((1, D), lambda i: (0, 0)),
        ],
        out_specs=pl.BlockSpec((_BS, D), lambda i: (i, 0)),
        out_shape=jax.ShapeDtypeStruct((B, D), jnp.float32),
    )(gp, pf, gsm, w4, b2)


def kernel(cell_type, dose, time, batch_id, E_cell, E_dose, E_time, E_batch, W, b):
    ct = cell_type.astype(jnp.int32)
    idx_m = (ct % H).reshape(B // CHUNK, CHUNK)
    pf = (ct >= H).astype(jnp.float32).reshape(B, 1)
    do = dose.astype(jnp.int32).reshape(B // CHUNK, CHUNK)
    ti = time.astype(jnp.int32).reshape(B // CHUNK, CHUNK)
    bi = batch_id.astype(jnp.int32).reshape(B // CHUNK, CHUNK)
    pairs = _tc_make_pairs(E_cell.T)
    gp = _sc_gather_pairs(pairs, idx_m)
    gsm = _sc_gather_small(do, ti, bi, E_dose, E_time, E_batch)
    w4 = W.reshape(4, D, D)
    return _tc_proj(gp, pf, gsm, w4, b.reshape(1, D))


# speedup vs baseline: 1.3428x; 1.0811x over previous
import functools

import jax
import jax.numpy as jnp
from jax import lax
from jax.experimental import pallas as pl
from jax.experimental.pallas import tpu as pltpu
from jax.experimental.pallas import tpu_sc as plsc

B = 16384
D = 64
COV_CELL = 1000000
H = 524288
LB = 1024
CHUNK = 128
_info = plsc.get_sparse_core_info()
NC, NS = _info.num_cores, _info.num_subcores
NW = NC * NS
BPW = B // NW
NCH = BPW // CHUNK

_mesh = plsc.VectorSubcoreMesh(core_axis_name="c", subcore_axis_name="s")


def _tc_pairs_body(a_ref, b_ref, w_ref, o_ref):
    ab = jnp.concatenate([a_ref[...], b_ref[...]], axis=0)
    o_ref[...] = lax.dot_general(ab, w_ref[...], (((0,), (0,)), ((), ())),
                                 preferred_element_type=jnp.float32)


def _tc_make_pairs(et2, wst):
    nblk = H // LB
    last = (COV_CELL - 1) // LB
    return pl.pallas_call(
        _tc_pairs_body,
        grid=(nblk,),
        compiler_params=pltpu.CompilerParams(fuse_transposed_lhs_in_matmul=True),
        in_specs=[
            pl.BlockSpec((D, LB), lambda g: (0, g)),
            pl.BlockSpec((D, LB), lambda g: (0, jnp.minimum(g + nblk, last))),
            pl.BlockSpec((2 * D, 2 * D), lambda g: (0, 0)),
        ],
        out_specs=pl.BlockSpec((LB, 2 * D), lambda g: (g, 0)),
        out_shape=jax.ShapeDtypeStruct((H, 2 * D), jnp.float32),
    )(et2, et2, wst)


@functools.partial(
    pl.kernel,
    mesh=_mesh,
    compiler_params=pltpu.CompilerParams(use_tc_tiling_on_sc=True),
    out_type=jax.ShapeDtypeStruct((B, 2 * D), jnp.float32),
    scratch_types=[
        pltpu.VMEM((NCH, CHUNK), jnp.int32),
        pltpu.VMEM((BPW, 2 * D), jnp.float32),
        pltpu.SemaphoreType.DMA,
    ],
)
def _sc_gather_pairs(tab_hbm, idx_hbm, out_hbm, idx_v, rows_v, gsem):
    wid = lax.axis_index("s") * NC + lax.axis_index("c")
    base = wid * BPW
    pltpu.sync_copy(idx_hbm.at[pl.ds(wid * NCH, NCH)], idx_v)
    gathers = []
    for k in range(NCH):
        gathers.append(pltpu.async_copy(
            tab_hbm.at[idx_v.at[k]], rows_v.at[pl.ds(k * CHUNK, CHUNK)], gsem))
    for g in gathers:
        g.wait()
    pltpu.sync_copy(rows_v, out_hbm.at[pl.ds(base, BPW)])


@functools.partial(
    pl.kernel,
    mesh=_mesh,
    compiler_params=pltpu.CompilerParams(use_tc_tiling_on_sc=False),
    out_type=jax.ShapeDtypeStruct((3, B, D), jnp.float32),
    scratch_types=[
        pltpu.VMEM((NCH, CHUNK), jnp.int32),
        pltpu.VMEM((BPW, D), jnp.float32),
        pltpu.VMEM((BPW, D), jnp.float32),
        pltpu.SemaphoreType.DMA,
        pltpu.SemaphoreType.DMA,
    ],
)
def _sc_gather_small(dose_hbm, time_hbm, bid_hbm,
                     ed_hbm, et_hbm, eb_hbm,
                     out_hbm, idx_v, rows_a, rows_b, gsem, osem):
    wid = lax.axis_index("s") * NC + lax.axis_index("c")
    base = wid * BPW
    cbase = wid * NCH

    idxs = (dose_hbm, time_hbm, bid_hbm)
    tabs = (ed_hbm, et_hbm, eb_hbm)
    bufs = (rows_a, rows_b)
    stores = [None, None]
    for j in range(3):
        buf = bufs[j % 2]
        if stores[j % 2] is not None:
            stores[j % 2].wait()
        pltpu.sync_copy(idxs[j].at[pl.ds(cbase, NCH)], idx_v)
        gathers = []
        for k in range(NCH):
            gathers.append(pltpu.async_copy(
                tabs[j].at[idx_v.at[k]], buf.at[pl.ds(k * CHUNK, CHUNK)], gsem))
        for g in gathers:
            g.wait()
        stores[j % 2] = pltpu.async_copy(
            buf, out_hbm.at[j, pl.ds(base, BPW)], osem)
    for s in stores:
        if s is not None:
            s.wait()


_BS = 2048


def _tc_proj_body(gp_ref, pf_ref, gsm_ref, w_ref, b_ref, o_ref):
    y0 = gp_ref[:, 0:D]
    y1 = gp_ref[:, D:2 * D]
    acc = y0 + pf_ref[...] * (y1 - y0)
    for j in range(3):
        acc = acc + jnp.dot(gsm_ref[j], w_ref[j],
                            preferred_element_type=jnp.float32)
    o_ref[...] = acc + b_ref[...]


def _tc_proj(gp, pf, gsm, w3, b2):
    return pl.pallas_call(
        _tc_proj_body,
        grid=(B // _BS,),
        in_specs=[
            pl.BlockSpec((_BS, 2 * D), lambda i: (i, 0)),
            pl.BlockSpec((_BS, 1), lambda i: (i, 0)),
            pl.BlockSpec((3, _BS, D), lambda i: (0, i, 0)),
            pl.BlockSpec((3, D, D), lambda i: (0, 0, 0)),
            pl.BlockSpec((1, D), lambda i: (0, 0)),
        ],
        out_specs=pl.BlockSpec((_BS, D), lambda i: (i, 0)),
        out_shape=jax.ShapeDtypeStruct((B, D), jnp.float32),
    )(gp, pf, gsm, w3, b2)


def kernel(cell_type, dose, time, batch_id, E_cell, E_dose, E_time, E_batch, W, b):
    ct = cell_type.astype(jnp.int32)
    idx_m = (ct % H).reshape(B // CHUNK, CHUNK)
    pf = (ct >= H).astype(jnp.float32).reshape(B, 1)
    do = dose.astype(jnp.int32).reshape(B // CHUNK, CHUNK)
    ti = time.astype(jnp.int32).reshape(B // CHUNK, CHUNK)
    bi = batch_id.astype(jnp.int32).reshape(B // CHUNK, CHUNK)
    w1 = W[0:D, :]
    z = jnp.zeros((D, D), jnp.float32)
    wst = jnp.block([[w1, z], [z, w1]])
    pairs = _tc_make_pairs(E_cell.T, wst)
    gp = _sc_gather_pairs(pairs, idx_m)
    gsm = _sc_gather_small(do, ti, bi, E_dose, E_time, E_batch)
    w3 = W[D:, :].reshape(3, D, D)
    return _tc_proj(gp, pf, gsm, w3, b.reshape(1, D))


# speedup vs baseline: 1.8582x; 1.3838x over previous
import functools

import jax
import jax.numpy as jnp
from jax import lax
from jax.experimental import pallas as pl
from jax.experimental.pallas import tpu as pltpu
from jax.experimental.pallas import tpu_sc as plsc

B = 16384
D = 64
COV_CELL = 1000000
H = 524288
LB = 2048
CHUNK = 128
_info = plsc.get_sparse_core_info()
NC, NS = _info.num_cores, _info.num_subcores
NW = NC * NS
BPW = B // NW
NCH = BPW // CHUNK

_mesh = plsc.VectorSubcoreMesh(core_axis_name="c", subcore_axis_name="s")


def _tc_pairs_body(a_ref, b_ref, w_ref, o_ref):
    ab = jnp.concatenate([a_ref[...], b_ref[...]], axis=0)
    o_ref[...] = lax.dot_general(ab, w_ref[...], (((0,), (0,)), ((), ())),
                                 preferred_element_type=jnp.float32)


def _tc_make_pairs(et2, wst):
    nblk = H // LB
    last = (COV_CELL - 1) // LB
    return pl.pallas_call(
        _tc_pairs_body,
        grid=(nblk,),
        compiler_params=pltpu.CompilerParams(fuse_transposed_lhs_in_matmul=True),
        in_specs=[
            pl.BlockSpec((D, LB), lambda g: (0, g)),
            pl.BlockSpec((D, LB), lambda g: (0, jnp.minimum(g + nblk, last))),
            pl.BlockSpec((2 * D, 2 * D), lambda g: (0, 0)),
        ],
        out_specs=pl.BlockSpec((LB, 2 * D), lambda g: (g, 0)),
        out_shape=jax.ShapeDtypeStruct((H, 2 * D), jnp.float32),
    )(et2, et2, wst)


@functools.partial(
    pl.kernel,
    mesh=_mesh,
    compiler_params=pltpu.CompilerParams(use_tc_tiling_on_sc=True),
    out_type=jax.ShapeDtypeStruct((B, 2 * D), jnp.float32),
    scratch_types=[
        pltpu.VMEM((NCH, CHUNK), jnp.int32),
        pltpu.VMEM((BPW, 2 * D), jnp.float32),
        pltpu.SemaphoreType.DMA,
    ],
)
def _sc_gather_pairs(tab_hbm, idx_hbm, out_hbm, idx_v, rows_v, gsem):
    wid = lax.axis_index("s") * NC + lax.axis_index("c")
    base = wid * BPW
    pltpu.sync_copy(idx_hbm.at[pl.ds(wid * NCH, NCH)], idx_v)
    gathers = []
    for k in range(NCH):
        gathers.append(pltpu.async_copy(
            tab_hbm.at[idx_v.at[k]], rows_v.at[pl.ds(k * CHUNK, CHUNK)], gsem))
    for g in gathers:
        g.wait()
    pltpu.sync_copy(rows_v, out_hbm.at[pl.ds(base, BPW)])


@functools.partial(
    pl.kernel,
    mesh=_mesh,
    compiler_params=pltpu.CompilerParams(use_tc_tiling_on_sc=False),
    out_type=jax.ShapeDtypeStruct((3, B, D), jnp.float32),
    scratch_types=[
        pltpu.VMEM((NCH, CHUNK), jnp.int32),
        pltpu.VMEM((BPW, D), jnp.float32),
        pltpu.VMEM((BPW, D), jnp.float32),
        pltpu.SemaphoreType.DMA,
        pltpu.SemaphoreType.DMA,
    ],
)
def _sc_gather_small(dose_hbm, time_hbm, bid_hbm,
                     ed_hbm, et_hbm, eb_hbm,
                     out_hbm, idx_v, rows_a, rows_b, gsem, osem):
    wid = lax.axis_index("s") * NC + lax.axis_index("c")
    base = wid * BPW
    cbase = wid * NCH

    idxs = (dose_hbm, time_hbm, bid_hbm)
    tabs = (ed_hbm, et_hbm, eb_hbm)
    bufs = (rows_a, rows_b)
    stores = [None, None]
    for j in range(3):
        buf = bufs[j % 2]
        if stores[j % 2] is not None:
            stores[j % 2].wait()
        pltpu.sync_copy(idxs[j].at[pl.ds(cbase, NCH)], idx_v)
        gathers = []
        for k in range(NCH):
            gathers.append(pltpu.async_copy(
                tabs[j].at[idx_v.at[k]], buf.at[pl.ds(k * CHUNK, CHUNK)], gsem))
        for g in gathers:
            g.wait()
        stores[j % 2] = pltpu.async_copy(
            buf, out_hbm.at[j, pl.ds(base, BPW)], osem)
    for s in stores:
        if s is not None:
            s.wait()


_BS = 2048


def _tc_proj_body(gp_ref, pf_ref, gsm_ref, w_ref, b_ref, o_ref):
    y0 = gp_ref[:, 0:D]
    y1 = gp_ref[:, D:2 * D]
    acc = y0 + pf_ref[...] * (y1 - y0)
    for j in range(3):
        acc = acc + jnp.dot(gsm_ref[j], w_ref[j],
                            preferred_element_type=jnp.float32)
    o_ref[...] = acc + b_ref[...]


def _tc_proj(gp, pf, gsm, w3, b2):
    return pl.pallas_call(
        _tc_proj_body,
        grid=(B // _BS,),
        in_specs=[
            pl.BlockSpec((_BS, 2 * D), lambda i: (i, 0)),
            pl.BlockSpec((_BS, 1), lambda i: (i, 0)),
            pl.BlockSpec((3, _BS, D), lambda i: (0, i, 0)),
            pl.BlockSpec((3, D, D), lambda i: (0, 0, 0)),
            pl.BlockSpec((1, D), lambda i: (0, 0)),
        ],
        out_specs=pl.BlockSpec((_BS, D), lambda i: (i, 0)),
        out_shape=jax.ShapeDtypeStruct((B, D), jnp.float32),
    )(gp, pf, gsm, w3, b2)


def kernel(cell_type, dose, time, batch_id, E_cell, E_dose, E_time, E_batch, W, b):
    ct = cell_type.astype(jnp.int32)
    idx_m = (ct % H).reshape(B // CHUNK, CHUNK)
    pf = (ct >= H).astype(jnp.float32).reshape(B, 1)
    do = dose.astype(jnp.int32).reshape(B // CHUNK, CHUNK)
    ti = time.astype(jnp.int32).reshape(B // CHUNK, CHUNK)
    bi = batch_id.astype(jnp.int32).reshape(B // CHUNK, CHUNK)
    w1 = W[0:D, :]
    z = jnp.zeros((D, D), jnp.float32)
    wst = jnp.block([[w1, z], [z, w1]])
    pairs = _tc_make_pairs(E_cell.T, wst)
    gp = _sc_gather_pairs(pairs, idx_m)
    gsm = _sc_gather_small(do, ti, bi, E_dose, E_time, E_batch)
    w3 = W[D:, :].reshape(3, D, D)
    return _tc_proj(gp, pf, gsm, w3, b.reshape(1, D))


# speedup vs baseline: 2.3581x; 1.2690x over previous
import functools

import jax
import jax.numpy as jnp
from jax import lax
from jax.experimental import pallas as pl
from jax.experimental.pallas import tpu as pltpu
from jax.experimental.pallas import tpu_sc as plsc

B = 16384
D = 64
COV_CELL = 1000000
H = 524288
LB = 4096
CHUNK = 128
_info = plsc.get_sparse_core_info()
NC, NS = _info.num_cores, _info.num_subcores
NW = NC * NS
BPW = B // NW
NCH = BPW // CHUNK

_mesh = plsc.VectorSubcoreMesh(core_axis_name="c", subcore_axis_name="s")


def _tc_pairs_body(a_ref, b_ref, w_ref, o_ref):
    ab = jnp.concatenate([a_ref[...], b_ref[...]], axis=0)
    o_ref[...] = lax.dot_general(ab, w_ref[...], (((0,), (0,)), ((), ())),
                                 preferred_element_type=jnp.float32)


def _tc_make_pairs(et2, wst):
    nblk = H // LB
    last = (COV_CELL - 1) // LB
    return pl.pallas_call(
        _tc_pairs_body,
        grid=(nblk,),
        compiler_params=pltpu.CompilerParams(fuse_transposed_lhs_in_matmul=True),
        in_specs=[
            pl.BlockSpec((D, LB), lambda g: (0, g)),
            pl.BlockSpec((D, LB), lambda g: (0, jnp.minimum(g + nblk, last))),
            pl.BlockSpec((2 * D, 2 * D), lambda g: (0, 0)),
        ],
        out_specs=pl.BlockSpec((LB, 2 * D), lambda g: (g, 0)),
        out_shape=jax.ShapeDtypeStruct((H, 2 * D), jnp.float32),
    )(et2, et2, wst)


@functools.partial(
    pl.kernel,
    mesh=_mesh,
    compiler_params=pltpu.CompilerParams(use_tc_tiling_on_sc=True),
    out_type=jax.ShapeDtypeStruct((B, 2 * D), jnp.float32),
    scratch_types=[
        pltpu.VMEM((NCH, CHUNK), jnp.int32),
        pltpu.VMEM((BPW, 2 * D), jnp.float32),
        pltpu.SemaphoreType.DMA,
    ],
)
def _sc_gather_pairs(tab_hbm, idx_hbm, out_hbm, idx_v, rows_v, gsem):
    wid = lax.axis_index("s") * NC + lax.axis_index("c")
    base = wid * BPW
    pltpu.sync_copy(idx_hbm.at[pl.ds(wid * NCH, NCH)], idx_v)
    gathers = []
    for k in range(NCH):
        gathers.append(pltpu.async_copy(
            tab_hbm.at[idx_v.at[k]], rows_v.at[pl.ds(k * CHUNK, CHUNK)], gsem))
    for g in gathers:
        g.wait()
    pltpu.sync_copy(rows_v, out_hbm.at[pl.ds(base, BPW)])


@functools.partial(
    pl.kernel,
    mesh=_mesh,
    compiler_params=pltpu.CompilerParams(use_tc_tiling_on_sc=False),
    out_type=jax.ShapeDtypeStruct((3, B, D), jnp.float32),
    scratch_types=[
        pltpu.VMEM((NCH, CHUNK), jnp.int32),
        pltpu.VMEM((BPW, D), jnp.float32),
        pltpu.VMEM((BPW, D), jnp.float32),
        pltpu.SemaphoreType.DMA,
        pltpu.SemaphoreType.DMA,
    ],
)
def _sc_gather_small(dose_hbm, time_hbm, bid_hbm,
                     ed_hbm, et_hbm, eb_hbm,
                     out_hbm, idx_v, rows_a, rows_b, gsem, osem):
    wid = lax.axis_index("s") * NC + lax.axis_index("c")
    base = wid * BPW
    cbase = wid * NCH

    idxs = (dose_hbm, time_hbm, bid_hbm)
    tabs = (ed_hbm, et_hbm, eb_hbm)
    bufs = (rows_a, rows_b)
    stores = [None, None]
    for j in range(3):
        buf = bufs[j % 2]
        if stores[j % 2] is not None:
            stores[j % 2].wait()
        pltpu.sync_copy(idxs[j].at[pl.ds(cbase, NCH)], idx_v)
        gathers = []
        for k in range(NCH):
            gathers.append(pltpu.async_copy(
                tabs[j].at[idx_v.at[k]], buf.at[pl.ds(k * CHUNK, CHUNK)], gsem))
        for g in gathers:
            g.wait()
        stores[j % 2] = pltpu.async_copy(
            buf, out_hbm.at[j, pl.ds(base, BPW)], osem)
    for s in stores:
        if s is not None:
            s.wait()


_BS = 2048


def _tc_proj_body(gp_ref, pf_ref, gsm_ref, w_ref, b_ref, o_ref):
    y0 = gp_ref[:, 0:D]
    y1 = gp_ref[:, D:2 * D]
    acc = y0 + pf_ref[...] * (y1 - y0)
    for j in range(3):
        acc = acc + jnp.dot(gsm_ref[j], w_ref[j],
                            preferred_element_type=jnp.float32)
    o_ref[...] = acc + b_ref[...]


def _tc_proj(gp, pf, gsm, w3, b2):
    return pl.pallas_call(
        _tc_proj_body,
        grid=(B // _BS,),
        in_specs=[
            pl.BlockSpec((_BS, 2 * D), lambda i: (i, 0)),
            pl.BlockSpec((_BS, 1), lambda i: (i, 0)),
            pl.BlockSpec((3, _BS, D), lambda i: (0, i, 0)),
            pl.BlockSpec((3, D, D), lambda i: (0, 0, 0)),
            pl.BlockSpec((1, D), lambda i: (0, 0)),
        ],
        out_specs=pl.BlockSpec((_BS, D), lambda i: (i, 0)),
        out_shape=jax.ShapeDtypeStruct((B, D), jnp.float32),
    )(gp, pf, gsm, w3, b2)


def kernel(cell_type, dose, time, batch_id, E_cell, E_dose, E_time, E_batch, W, b):
    ct = cell_type.astype(jnp.int32)
    idx_m = (ct % H).reshape(B // CHUNK, CHUNK)
    pf = (ct >= H).astype(jnp.float32).reshape(B, 1)
    do = dose.astype(jnp.int32).reshape(B // CHUNK, CHUNK)
    ti = time.astype(jnp.int32).reshape(B // CHUNK, CHUNK)
    bi = batch_id.astype(jnp.int32).reshape(B // CHUNK, CHUNK)
    w1 = W[0:D, :]
    z = jnp.zeros((D, D), jnp.float32)
    wst = jnp.block([[w1, z], [z, w1]])
    pairs = _tc_make_pairs(E_cell.T, wst)
    gp = _sc_gather_pairs(pairs, idx_m)
    gsm = _sc_gather_small(do, ti, bi, E_dose, E_time, E_batch)
    w3 = W[D:, :].reshape(3, D, D)
    return _tc_proj(gp, pf, gsm, w3, b.reshape(1, D))


# speedup vs baseline: 2.6221x; 1.1120x over previous
import functools

import jax
import jax.numpy as jnp
from jax import lax
from jax.experimental import pallas as pl
from jax.experimental.pallas import tpu as pltpu
from jax.experimental.pallas import tpu_sc as plsc

B = 16384
D = 64
COV_CELL = 1000000
H = 524288
LB = 8192
CHUNK = 128
_info = plsc.get_sparse_core_info()
NC, NS = _info.num_cores, _info.num_subcores
NW = NC * NS
BPW = B // NW
NCH = BPW // CHUNK

_mesh = plsc.VectorSubcoreMesh(core_axis_name="c", subcore_axis_name="s")


def _tc_pairs_body(a_ref, b_ref, w_ref, o_ref):
    ab = jnp.concatenate([a_ref[...], b_ref[...]], axis=0)
    o_ref[...] = lax.dot_general(ab, w_ref[...], (((0,), (0,)), ((), ())),
                                 preferred_element_type=jnp.float32)


def _tc_make_pairs(et2, wst):
    nblk = H // LB
    last = (COV_CELL - 1) // LB
    return pl.pallas_call(
        _tc_pairs_body,
        grid=(nblk,),
        compiler_params=pltpu.CompilerParams(fuse_transposed_lhs_in_matmul=True),
        in_specs=[
            pl.BlockSpec((D, LB), lambda g: (0, g)),
            pl.BlockSpec((D, LB), lambda g: (0, jnp.minimum(g + nblk, last))),
            pl.BlockSpec((2 * D, 2 * D), lambda g: (0, 0)),
        ],
        out_specs=pl.BlockSpec((LB, 2 * D), lambda g: (g, 0)),
        out_shape=jax.ShapeDtypeStruct((H, 2 * D), jnp.float32),
    )(et2, et2, wst)


@functools.partial(
    pl.kernel,
    mesh=_mesh,
    compiler_params=pltpu.CompilerParams(use_tc_tiling_on_sc=True),
    out_type=jax.ShapeDtypeStruct((B, 2 * D), jnp.float32),
    scratch_types=[
        pltpu.VMEM((NCH, CHUNK), jnp.int32),
        pltpu.VMEM((BPW, 2 * D), jnp.float32),
        pltpu.SemaphoreType.DMA,
    ],
)
def _sc_gather_pairs(tab_hbm, idx_hbm, out_hbm, idx_v, rows_v, gsem):
    wid = lax.axis_index("s") * NC + lax.axis_index("c")
    base = wid * BPW
    pltpu.sync_copy(idx_hbm.at[pl.ds(wid * NCH, NCH)], idx_v)
    gathers = []
    for k in range(NCH):
        gathers.append(pltpu.async_copy(
            tab_hbm.at[idx_v.at[k]], rows_v.at[pl.ds(k * CHUNK, CHUNK)], gsem))
    for g in gathers:
        g.wait()
    pltpu.sync_copy(rows_v, out_hbm.at[pl.ds(base, BPW)])


@functools.partial(
    pl.kernel,
    mesh=_mesh,
    compiler_params=pltpu.CompilerParams(use_tc_tiling_on_sc=False),
    out_type=jax.ShapeDtypeStruct((3, B, D), jnp.float32),
    scratch_types=[
        pltpu.VMEM((NCH, CHUNK), jnp.int32),
        pltpu.VMEM((BPW, D), jnp.float32),
        pltpu.VMEM((BPW, D), jnp.float32),
        pltpu.SemaphoreType.DMA,
        pltpu.SemaphoreType.DMA,
    ],
)
def _sc_gather_small(dose_hbm, time_hbm, bid_hbm,
                     ed_hbm, et_hbm, eb_hbm,
                     out_hbm, idx_v, rows_a, rows_b, gsem, osem):
    wid = lax.axis_index("s") * NC + lax.axis_index("c")
    base = wid * BPW
    cbase = wid * NCH

    idxs = (dose_hbm, time_hbm, bid_hbm)
    tabs = (ed_hbm, et_hbm, eb_hbm)
    bufs = (rows_a, rows_b)
    stores = [None, None]
    for j in range(3):
        buf = bufs[j % 2]
        if stores[j % 2] is not None:
            stores[j % 2].wait()
        pltpu.sync_copy(idxs[j].at[pl.ds(cbase, NCH)], idx_v)
        gathers = []
        for k in range(NCH):
            gathers.append(pltpu.async_copy(
                tabs[j].at[idx_v.at[k]], buf.at[pl.ds(k * CHUNK, CHUNK)], gsem))
        for g in gathers:
            g.wait()
        stores[j % 2] = pltpu.async_copy(
            buf, out_hbm.at[j, pl.ds(base, BPW)], osem)
    for s in stores:
        if s is not None:
            s.wait()


_BS = 2048


def _tc_proj_body(gp_ref, pf_ref, gsm_ref, w_ref, b_ref, o_ref):
    y0 = gp_ref[:, 0:D]
    y1 = gp_ref[:, D:2 * D]
    acc = y0 + pf_ref[...] * (y1 - y0)
    for j in range(3):
        acc = acc + jnp.dot(gsm_ref[j], w_ref[j],
                            preferred_element_type=jnp.float32)
    o_ref[...] = acc + b_ref[...]


def _tc_proj(gp, pf, gsm, w3, b2):
    return pl.pallas_call(
        _tc_proj_body,
        grid=(B // _BS,),
        in_specs=[
            pl.BlockSpec((_BS, 2 * D), lambda i: (i, 0)),
            pl.BlockSpec((_BS, 1), lambda i: (i, 0)),
            pl.BlockSpec((3, _BS, D), lambda i: (0, i, 0)),
            pl.BlockSpec((3, D, D), lambda i: (0, 0, 0)),
            pl.BlockSpec((1, D), lambda i: (0, 0)),
        ],
        out_specs=pl.BlockSpec((_BS, D), lambda i: (i, 0)),
        out_shape=jax.ShapeDtypeStruct((B, D), jnp.float32),
    )(gp, pf, gsm, w3, b2)


def kernel(cell_type, dose, time, batch_id, E_cell, E_dose, E_time, E_batch, W, b):
    ct = cell_type.astype(jnp.int32)
    idx_m = (ct % H).reshape(B // CHUNK, CHUNK)
    pf = (ct >= H).astype(jnp.float32).reshape(B, 1)
    do = dose.astype(jnp.int32).reshape(B // CHUNK, CHUNK)
    ti = time.astype(jnp.int32).reshape(B // CHUNK, CHUNK)
    bi = batch_id.astype(jnp.int32).reshape(B // CHUNK, CHUNK)
    w1 = W[0:D, :]
    z = jnp.zeros((D, D), jnp.float32)
    wst = jnp.block([[w1, z], [z, w1]])
    pairs = _tc_make_pairs(E_cell.T, wst)
    gp = _sc_gather_pairs(pairs, idx_m)
    gsm = _sc_gather_small(do, ti, bi, E_dose, E_time, E_batch)
    w3 = W[D:, :].reshape(3, D, D)
    return _tc_proj(gp, pf, gsm, w3, b.reshape(1, D))


# speedup vs baseline: 2.6897x; 1.0258x over previous
import functools

import jax
import jax.numpy as jnp
from jax import lax
from jax.experimental import pallas as pl
from jax.experimental.pallas import tpu as pltpu
from jax.experimental.pallas import tpu_sc as plsc

B = 16384
D = 64
COV_CELL = 1000000
H = 524288
LB = 16384
CHUNK = 128
_info = plsc.get_sparse_core_info()
NC, NS = _info.num_cores, _info.num_subcores
NW = NC * NS
BPW = B // NW
NCH = BPW // CHUNK

_mesh = plsc.VectorSubcoreMesh(core_axis_name="c", subcore_axis_name="s")


def _tc_pairs_body(a_ref, b_ref, w_ref, o_ref):
    ab = jnp.concatenate([a_ref[...], b_ref[...]], axis=0)
    o_ref[...] = lax.dot_general(ab, w_ref[...], (((0,), (0,)), ((), ())),
                                 preferred_element_type=jnp.float32)


def _tc_make_pairs(et2, wst):
    nblk = H // LB
    last = (COV_CELL - 1) // LB
    return pl.pallas_call(
        _tc_pairs_body,
        grid=(nblk,),
        compiler_params=pltpu.CompilerParams(fuse_transposed_lhs_in_matmul=True),
        in_specs=[
            pl.BlockSpec((D, LB), lambda g: (0, g)),
            pl.BlockSpec((D, LB), lambda g: (0, jnp.minimum(g + nblk, last))),
            pl.BlockSpec((2 * D, 2 * D), lambda g: (0, 0)),
        ],
        out_specs=pl.BlockSpec((LB, 2 * D), lambda g: (g, 0)),
        out_shape=jax.ShapeDtypeStruct((H, 2 * D), jnp.float32),
    )(et2, et2, wst)


@functools.partial(
    pl.kernel,
    mesh=_mesh,
    compiler_params=pltpu.CompilerParams(use_tc_tiling_on_sc=True),
    out_type=jax.ShapeDtypeStruct((B, 2 * D), jnp.float32),
    scratch_types=[
        pltpu.VMEM((NCH, CHUNK), jnp.int32),
        pltpu.VMEM((BPW, 2 * D), jnp.float32),
        pltpu.SemaphoreType.DMA,
    ],
)
def _sc_gather_pairs(tab_hbm, idx_hbm, out_hbm, idx_v, rows_v, gsem):
    wid = lax.axis_index("s") * NC + lax.axis_index("c")
    base = wid * BPW
    pltpu.sync_copy(idx_hbm.at[pl.ds(wid * NCH, NCH)], idx_v)
    gathers = []
    for k in range(NCH):
        gathers.append(pltpu.async_copy(
            tab_hbm.at[idx_v.at[k]], rows_v.at[pl.ds(k * CHUNK, CHUNK)], gsem))
    for g in gathers:
        g.wait()
    pltpu.sync_copy(rows_v, out_hbm.at[pl.ds(base, BPW)])


@functools.partial(
    pl.kernel,
    mesh=_mesh,
    compiler_params=pltpu.CompilerParams(use_tc_tiling_on_sc=False),
    out_type=jax.ShapeDtypeStruct((3, B, D), jnp.float32),
    scratch_types=[
        pltpu.VMEM((NCH, CHUNK), jnp.int32),
        pltpu.VMEM((BPW, D), jnp.float32),
        pltpu.VMEM((BPW, D), jnp.float32),
        pltpu.SemaphoreType.DMA,
        pltpu.SemaphoreType.DMA,
    ],
)
def _sc_gather_small(dose_hbm, time_hbm, bid_hbm,
                     ed_hbm, et_hbm, eb_hbm,
                     out_hbm, idx_v, rows_a, rows_b, gsem, osem):
    wid = lax.axis_index("s") * NC + lax.axis_index("c")
    base = wid * BPW
    cbase = wid * NCH

    idxs = (dose_hbm, time_hbm, bid_hbm)
    tabs = (ed_hbm, et_hbm, eb_hbm)
    bufs = (rows_a, rows_b)
    stores = [None, None]
    for j in range(3):
        buf = bufs[j % 2]
        if stores[j % 2] is not None:
            stores[j % 2].wait()
        pltpu.sync_copy(idxs[j].at[pl.ds(cbase, NCH)], idx_v)
        gathers = []
        for k in range(NCH):
            gathers.append(pltpu.async_copy(
                tabs[j].at[idx_v.at[k]], buf.at[pl.ds(k * CHUNK, CHUNK)], gsem))
        for g in gathers:
            g.wait()
        stores[j % 2] = pltpu.async_copy(
            buf, out_hbm.at[j, pl.ds(base, BPW)], osem)
    for s in stores:
        if s is not None:
            s.wait()


_BS = 2048


def _tc_proj_body(gp_ref, pf_ref, gsm_ref, w_ref, b_ref, o_ref):
    y0 = gp_ref[:, 0:D]
    y1 = gp_ref[:, D:2 * D]
    acc = y0 + pf_ref[...] * (y1 - y0)
    for j in range(3):
        acc = acc + jnp.dot(gsm_ref[j], w_ref[j],
                            preferred_element_type=jnp.float32)
    o_ref[...] = acc + b_ref[...]


def _tc_proj(gp, pf, gsm, w3, b2):
    return pl.pallas_call(
        _tc_proj_body,
        grid=(B // _BS,),
        in_specs=[
            pl.BlockSpec((_BS, 2 * D), lambda i: (i, 0)),
            pl.BlockSpec((_BS, 1), lambda i: (i, 0)),
            pl.BlockSpec((3, _BS, D), lambda i: (0, i, 0)),
            pl.BlockSpec((3, D, D), lambda i: (0, 0, 0)),
            pl.BlockSpec((1, D), lambda i: (0, 0)),
        ],
        out_specs=pl.BlockSpec((_BS, D), lambda i: (i, 0)),
        out_shape=jax.ShapeDtypeStruct((B, D), jnp.float32),
    )(gp, pf, gsm, w3, b2)


def kernel(cell_type, dose, time, batch_id, E_cell, E_dose, E_time, E_batch, W, b):
    ct = cell_type.astype(jnp.int32)
    idx_m = (ct % H).reshape(B // CHUNK, CHUNK)
    pf = (ct >= H).astype(jnp.float32).reshape(B, 1)
    do = dose.astype(jnp.int32).reshape(B // CHUNK, CHUNK)
    ti = time.astype(jnp.int32).reshape(B // CHUNK, CHUNK)
    bi = batch_id.astype(jnp.int32).reshape(B // CHUNK, CHUNK)
    w1 = W[0:D, :]
    z = jnp.zeros((D, D), jnp.float32)
    wst = jnp.block([[w1, z], [z, w1]])
    pairs = _tc_make_pairs(E_cell.T, wst)
    gp = _sc_gather_pairs(pairs, idx_m)
    gsm = _sc_gather_small(do, ti, bi, E_dose, E_time, E_batch)
    w3 = W[D:, :].reshape(3, D, D)
    return _tc_proj(gp, pf, gsm, w3, b.reshape(1, D))


# speedup vs baseline: 2.9825x; 1.1089x over previous
import functools

import jax
import jax.numpy as jnp
from jax import lax
from jax.experimental import pallas as pl
from jax.experimental.pallas import tpu as pltpu
from jax.experimental.pallas import tpu_sc as plsc

B = 16384
D = 64
COV_CELL = 1000000
Q = 262144
H = 2 * Q
LB = 8192
CHUNK = 128
_info = plsc.get_sparse_core_info()
NC, NS = _info.num_cores, _info.num_subcores
NW = NC * NS
BPW = B // NW
NCH = BPW // CHUNK

_mesh = plsc.VectorSubcoreMesh(core_axis_name="c", subcore_axis_name="s")


def _bf16_hi(i32bits):
    return (i32bits + 0x7FFF + ((i32bits >> 16) & 1)) & jnp.int32(-65536)


def _tc_pack_body(a_ref, b_ref, c_ref, d_ref, w_ref, o_ref):
    abcd = jnp.concatenate(
        [a_ref[...], b_ref[...], c_ref[...], d_ref[...]], axis=0)
    y = lax.dot_general(abcd, w_ref[...], (((0,), (0,)), ((), ())),
                        preferred_element_type=jnp.float32)
    ilo = lax.bitcast_convert_type(y[:, 0:2 * D], jnp.int32)
    ihi = lax.bitcast_convert_type(y[:, 2 * D:4 * D], jnp.int32)
    lo16 = (_bf16_hi(ilo) >> 16) & jnp.int32(0xFFFF)
    o_ref[...] = lo16 | _bf16_hi(ihi)


def _tc_pack(et2, wst):
    nblk = Q // LB
    hb = H // LB
    last = (COV_CELL - 1) // LB
    return pl.pallas_call(
        _tc_pack_body,
        grid=(nblk,),
        in_specs=[
            pl.BlockSpec((D, LB), lambda g: (0, g)),
            pl.BlockSpec((D, LB), lambda g: (0, jnp.minimum(g + hb, last))),
            pl.BlockSpec((D, LB), lambda g: (0, g + nblk)),
            pl.BlockSpec((D, LB),
                         lambda g: (0, jnp.minimum(g + nblk + hb, last))),
            pl.BlockSpec((4 * D, 4 * D), lambda g: (0, 0)),
        ],
        out_specs=pl.BlockSpec((LB, 2 * D), lambda g: (g, 0)),
        out_shape=jax.ShapeDtypeStruct((Q, 2 * D), jnp.int32),
    )(et2, et2, et2, et2, wst)


@functools.partial(
    pl.kernel,
    mesh=_mesh,
    compiler_params=pltpu.CompilerParams(use_tc_tiling_on_sc=True),
    out_type=jax.ShapeDtypeStruct((B, 2 * D), jnp.int32),
    scratch_types=[
        pltpu.VMEM((NCH, CHUNK), jnp.int32),
        pltpu.VMEM((BPW, 2 * D), jnp.int32),
        pltpu.SemaphoreType.DMA,
    ],
)
def _sc_gather_packed(tab_hbm, idx_hbm, out_hbm, idx_v, rows_v, gsem):
    wid = lax.axis_index("s") * NC + lax.axis_index("c")
    base = wid * BPW
    pltpu.sync_copy(idx_hbm.at[pl.ds(wid * NCH, NCH)], idx_v)
    gathers = []
    for k in range(NCH):
        gathers.append(pltpu.async_copy(
            tab_hbm.at[idx_v.at[k]], rows_v.at[pl.ds(k * CHUNK, CHUNK)], gsem))
    for g in gathers:
        g.wait()
    pltpu.sync_copy(rows_v, out_hbm.at[pl.ds(base, BPW)])


@functools.partial(
    pl.kernel,
    mesh=_mesh,
    compiler_params=pltpu.CompilerParams(use_tc_tiling_on_sc=False),
    out_type=jax.ShapeDtypeStruct((3, B, D), jnp.float32),
    scratch_types=[
        pltpu.VMEM((NCH, CHUNK), jnp.int32),
        pltpu.VMEM((BPW, D), jnp.float32),
        pltpu.VMEM((BPW, D), jnp.float32),
        pltpu.SemaphoreType.DMA,
        pltpu.SemaphoreType.DMA,
    ],
)
def _sc_gather_small(dose_hbm, time_hbm, bid_hbm,
                     ed_hbm, et_hbm, eb_hbm,
                     out_hbm, idx_v, rows_a, rows_b, gsem, osem):
    wid = lax.axis_index("s") * NC + lax.axis_index("c")
    base = wid * BPW
    cbase = wid * NCH

    idxs = (dose_hbm, time_hbm, bid_hbm)
    tabs = (ed_hbm, et_hbm, eb_hbm)
    bufs = (rows_a, rows_b)
    stores = [None, None]
    for j in range(3):
        buf = bufs[j % 2]
        if stores[j % 2] is not None:
            stores[j % 2].wait()
        pltpu.sync_copy(idxs[j].at[pl.ds(cbase, NCH)], idx_v)
        gathers = []
        for k in range(NCH):
            gathers.append(pltpu.async_copy(
                tabs[j].at[idx_v.at[k]], buf.at[pl.ds(k * CHUNK, CHUNK)], gsem))
        for g in gathers:
            g.wait()
        stores[j % 2] = pltpu.async_copy(
            buf, out_hbm.at[j, pl.ds(base, BPW)], osem)
    for s in stores:
        if s is not None:
            s.wait()


_BS = 2048


def _tc_proj_body(gp_ref, pf_ref, qf_ref, gsm_ref, w_ref, b_ref, o_ref):
    wbits = gp_ref[...]
    flo = lax.bitcast_convert_type(jnp.left_shift(wbits, 16), jnp.float32)
    fhi = lax.bitcast_convert_type(wbits & jnp.int32(-65536), jnp.float32)
    yq = flo + qf_ref[...] * (fhi - flo)
    y0 = yq[:, 0:D]
    y1 = yq[:, D:2 * D]
    acc = y0 + pf_ref[...] * (y1 - y0)
    for j in range(3):
        acc = acc + jnp.dot(gsm_ref[j], w_ref[j],
                            preferred_element_type=jnp.float32)
    o_ref[...] = acc + b_ref[...]


def _tc_proj(gp, pf, qf, gsm, w3, b2):
    return pl.pallas_call(
        _tc_proj_body,
        grid=(B // _BS,),
        in_specs=[
            pl.BlockSpec((_BS, 2 * D), lambda i: (i, 0)),
            pl.BlockSpec((_BS, 1), lambda i: (i, 0)),
            pl.BlockSpec((_BS, 1), lambda i: (i, 0)),
            pl.BlockSpec((3, _BS, D), lambda i: (0, i, 0)),
            pl.BlockSpec((3, D, D), lambda i: (0, 0, 0)),
            pl.BlockSpec((1, D), lambda i: (0, 0)),
        ],
        out_specs=pl.BlockSpec((_BS, D), lambda i: (i, 0)),
        out_shape=jax.ShapeDtypeStruct((B, D), jnp.float32),
    )(gp, pf, qf, gsm, w3, b2)


def kernel(cell_type, dose, time, batch_id, E_cell, E_dose, E_time, E_batch, W, b):
    ct = cell_type.astype(jnp.int32)
    m = ct % H
    idx_mm = (m % Q).reshape(B // CHUNK, CHUNK)
    pf = (ct >= H).astype(jnp.float32).reshape(B, 1)
    qf = (m >= Q).astype(jnp.float32).reshape(B, 1)
    do = dose.astype(jnp.int32).reshape(B // CHUNK, CHUNK)
    ti = time.astype(jnp.int32).reshape(B // CHUNK, CHUNK)
    bi = batch_id.astype(jnp.int32).reshape(B // CHUNK, CHUNK)
    w1 = W[0:D, :]
    wst = jnp.kron(jnp.eye(4, dtype=jnp.float32), w1)
    packed = _tc_pack(E_cell.T, wst)
    gp = _sc_gather_packed(packed, idx_mm)
    gsm = _sc_gather_small(do, ti, bi, E_dose, E_time, E_batch)
    w3 = W[D:, :].reshape(3, D, D)
    return _tc_proj(gp, pf, qf, gsm, w3, b.reshape(1, D))


# speedup vs baseline: 3.0042x; 1.0073x over previous
import functools

import jax
import jax.numpy as jnp
from jax import lax
from jax.experimental import pallas as pl
from jax.experimental.pallas import tpu as pltpu
from jax.experimental.pallas import tpu_sc as plsc

B = 16384
D = 64
COV_CELL = 1000000
Q = 262144
H = 2 * Q
LB = 16384
CHUNK = 128
_info = plsc.get_sparse_core_info()
NC, NS = _info.num_cores, _info.num_subcores
NW = NC * NS
BPW = B // NW
NCH = BPW // CHUNK

_mesh = plsc.VectorSubcoreMesh(core_axis_name="c", subcore_axis_name="s")


def _bf16_hi(i32bits):
    return (i32bits + 0x7FFF + ((i32bits >> 16) & 1)) & jnp.int32(-65536)


def _tc_pack_body(a_ref, b_ref, c_ref, d_ref, w_ref, o_ref):
    abcd = jnp.concatenate(
        [a_ref[...], b_ref[...], c_ref[...], d_ref[...]], axis=0)
    y = lax.dot_general(abcd, w_ref[...], (((0,), (0,)), ((), ())),
                        preferred_element_type=jnp.float32)
    ilo = lax.bitcast_convert_type(y[:, 0:2 * D], jnp.int32)
    ihi = lax.bitcast_convert_type(y[:, 2 * D:4 * D], jnp.int32)
    lo16 = (_bf16_hi(ilo) >> 16) & jnp.int32(0xFFFF)
    o_ref[...] = lo16 | _bf16_hi(ihi)


def _tc_pack(et2, wst):
    nblk = Q // LB
    hb = H // LB
    last = (COV_CELL - 1) // LB
    return pl.pallas_call(
        _tc_pack_body,
        grid=(nblk,),
        in_specs=[
            pl.BlockSpec((D, LB), lambda g: (0, g)),
            pl.BlockSpec((D, LB), lambda g: (0, jnp.minimum(g + hb, last))),
            pl.BlockSpec((D, LB), lambda g: (0, g + nblk)),
            pl.BlockSpec((D, LB),
                         lambda g: (0, jnp.minimum(g + nblk + hb, last))),
            pl.BlockSpec((4 * D, 4 * D), lambda g: (0, 0)),
        ],
        out_specs=pl.BlockSpec((LB, 2 * D), lambda g: (g, 0)),
        out_shape=jax.ShapeDtypeStruct((Q, 2 * D), jnp.int32),
    )(et2, et2, et2, et2, wst)


@functools.partial(
    pl.kernel,
    mesh=_mesh,
    compiler_params=pltpu.CompilerParams(use_tc_tiling_on_sc=True),
    out_type=jax.ShapeDtypeStruct((B, 2 * D), jnp.int32),
    scratch_types=[
        pltpu.VMEM((NCH, CHUNK), jnp.int32),
        pltpu.VMEM((BPW, 2 * D), jnp.int32),
        pltpu.SemaphoreType.DMA,
    ],
)
def _sc_gather_packed(tab_hbm, idx_hbm, out_hbm, idx_v, rows_v, gsem):
    wid = lax.axis_index("s") * NC + lax.axis_index("c")
    base = wid * BPW
    pltpu.sync_copy(idx_hbm.at[pl.ds(wid * NCH, NCH)], idx_v)
    gathers = []
    for k in range(NCH):
        gathers.append(pltpu.async_copy(
            tab_hbm.at[idx_v.at[k]], rows_v.at[pl.ds(k * CHUNK, CHUNK)], gsem))
    for g in gathers:
        g.wait()
    pltpu.sync_copy(rows_v, out_hbm.at[pl.ds(base, BPW)])


@functools.partial(
    pl.kernel,
    mesh=_mesh,
    compiler_params=pltpu.CompilerParams(use_tc_tiling_on_sc=False),
    out_type=jax.ShapeDtypeStruct((3, B, D), jnp.float32),
    scratch_types=[
        pltpu.VMEM((NCH, CHUNK), jnp.int32),
        pltpu.VMEM((BPW, D), jnp.float32),
        pltpu.VMEM((BPW, D), jnp.float32),
        pltpu.SemaphoreType.DMA,
        pltpu.SemaphoreType.DMA,
    ],
)
def _sc_gather_small(dose_hbm, time_hbm, bid_hbm,
                     ed_hbm, et_hbm, eb_hbm,
                     out_hbm, idx_v, rows_a, rows_b, gsem, osem):
    wid = lax.axis_index("s") * NC + lax.axis_index("c")
    base = wid * BPW
    cbase = wid * NCH

    idxs = (dose_hbm, time_hbm, bid_hbm)
    tabs = (ed_hbm, et_hbm, eb_hbm)
    bufs = (rows_a, rows_b)
    stores = [None, None]
    for j in range(3):
        buf = bufs[j % 2]
        if stores[j % 2] is not None:
            stores[j % 2].wait()
        pltpu.sync_copy(idxs[j].at[pl.ds(cbase, NCH)], idx_v)
        gathers = []
        for k in range(NCH):
            gathers.append(pltpu.async_copy(
                tabs[j].at[idx_v.at[k]], buf.at[pl.ds(k * CHUNK, CHUNK)], gsem))
        for g in gathers:
            g.wait()
        stores[j % 2] = pltpu.async_copy(
            buf, out_hbm.at[j, pl.ds(base, BPW)], osem)
    for s in stores:
        if s is not None:
            s.wait()


_BS = 2048


def _tc_proj_body(gp_ref, pf_ref, qf_ref, gsm_ref, w_ref, b_ref, o_ref):
    wbits = gp_ref[...]
    flo = lax.bitcast_convert_type(jnp.left_shift(wbits, 16), jnp.float32)
    fhi = lax.bitcast_convert_type(wbits & jnp.int32(-65536), jnp.float32)
    yq = flo + qf_ref[...] * (fhi - flo)
    y0 = yq[:, 0:D]
    y1 = yq[:, D:2 * D]
    acc = y0 + pf_ref[...] * (y1 - y0)
    for j in range(3):
        acc = acc + jnp.dot(gsm_ref[j], w_ref[j],
                            preferred_element_type=jnp.float32)
    o_ref[...] = acc + b_ref[...]


def _tc_proj(gp, pf, qf, gsm, w3, b2):
    return pl.pallas_call(
        _tc_proj_body,
        grid=(B // _BS,),
        in_specs=[
            pl.BlockSpec((_BS, 2 * D), lambda i: (i, 0)),
            pl.BlockSpec((_BS, 1), lambda i: (i, 0)),
            pl.BlockSpec((_BS, 1), lambda i: (i, 0)),
            pl.BlockSpec((3, _BS, D), lambda i: (0, i, 0)),
            pl.BlockSpec((3, D, D), lambda i: (0, 0, 0)),
            pl.BlockSpec((1, D), lambda i: (0, 0)),
        ],
        out_specs=pl.BlockSpec((_BS, D), lambda i: (i, 0)),
        out_shape=jax.ShapeDtypeStruct((B, D), jnp.float32),
    )(gp, pf, qf, gsm, w3, b2)


def kernel(cell_type, dose, time, batch_id, E_cell, E_dose, E_time, E_batch, W, b):
    ct = cell_type.astype(jnp.int32)
    m = ct % H
    idx_mm = (m % Q).reshape(B // CHUNK, CHUNK)
    pf = (ct >= H).astype(jnp.float32).reshape(B, 1)
    qf = (m >= Q).astype(jnp.float32).reshape(B, 1)
    do = dose.astype(jnp.int32).reshape(B // CHUNK, CHUNK)
    ti = time.astype(jnp.int32).reshape(B // CHUNK, CHUNK)
    bi = batch_id.astype(jnp.int32).reshape(B // CHUNK, CHUNK)
    w1 = W[0:D, :]
    wst = jnp.kron(jnp.eye(4, dtype=jnp.float32), w1)
    packed = _tc_pack(E_cell.T, wst)
    gp = _sc_gather_packed(packed, idx_mm)
    gsm = _sc_gather_small(do, ti, bi, E_dose, E_time, E_batch)
    w3 = W[D:, :].reshape(3, D, D)
    return _tc_proj(gp, pf, qf, gsm, w3, b.reshape(1, D))
